# baseline (speedup 1.0000x reference)
import functools

import jax
import jax.numpy as jnp
from jax import lax
from jax.experimental import pallas as pl
from jax.experimental.pallas import tpu as pltpu
from jax.experimental.pallas import tpu_sc as plsc

N = 10000
E = 320000
D_NODE = 128
D_EDGE = 16
H = 64

NC = 2
NS = 16
LANES = 16
NW = NC * NS
CHUNK = 9984
TAIL = E - NW * CHUNK
TAIL_OFF = NW * CHUNK
UNROLL = 8
CH_G = 19968
TAIL_G = E - NS * CH_G
TAIL_G_OFF = NS * CH_G


def _degree_body(idx_hbm, out_hbm, flat_hbm, idx_v, idx_t, idx_r, idx_rt,
                 ones_v, zeros_v, shared):
    c = lax.axis_index("c")
    s = lax.axis_index("s")
    wid = s * NC + c
    base = wid * CHUNK
    last = wid == NW - 1
    pltpu.sync_copy(idx_hbm.at[:, pl.ds(base, CHUNK)], idx_v)

    @pl.when(last)
    def _copy_tail():
        pltpu.sync_copy(idx_hbm.at[:, pl.ds(TAIL_OFF, TAIL)], idx_t)

    zeros = jnp.zeros((LANES,), jnp.float32)
    ones = jnp.ones((LANES,), jnp.float32)

    @pl.when(s == 0)
    def _zero_shared():
        def zbody(i, carry):
            zeros_v[pl.ds(i * LANES, LANES)] = zeros
            return carry
        lax.fori_loop(0, N // LANES, zbody, 0)
        pltpu.sync_copy(zeros_v, shared)

    def fbody(i, carry):
        ones_v[pl.ds(i * LANES, LANES)] = ones
        return carry
    lax.fori_loop(0, CHUNK // LANES, fbody, 0)

    def make_extract(src, dst, r):
        def xbody(i, carry):
            for k in range(UNROLL):
                off = i * (LANES * UNROLL) + k * LANES
                dst[pl.ds(off, LANES)] = (
                    src[r, pl.ds(off, LANES)].reshape((LANES,)))
            return carry
        return xbody

    plsc.subcore_barrier()
    for r in range(2):
        lax.fori_loop(0, CHUNK // (LANES * UNROLL),
                      make_extract(idx_v, idx_r, r), 0)
        pltpu.sync_copy(idx_r, flat_hbm.at[pl.ds(r * E + base, CHUNK)])
        pltpu.sync_copy(ones_v, shared.at[idx_r], add=True)

    @pl.when(last)
    def _scatter_tail():
        for r in range(2):
            lax.fori_loop(0, TAIL // (LANES * UNROLL),
                          make_extract(idx_t, idx_rt, r), 0)
            pltpu.sync_copy(idx_rt,
                            flat_hbm.at[pl.ds(r * E + TAIL_OFF, TAIL)])
            pltpu.sync_copy(ones_v.at[pl.ds(0, TAIL)],
                            shared.at[idx_rt], add=True)

    plsc.subcore_barrier()

    @pl.when(s == 0)
    def _flush():
        pltpu.sync_copy(shared, out_hbm.at[c])


def _gather_body(table_hbm, idx_hbm, out_hbm, table_v, idx_v, out_v):
    c = lax.axis_index("c")
    s = lax.axis_index("s")
    wid = s * NC + c
    row = wid // NS
    t = wid % NS
    base = t * CH_G
    last = t == NS - 1
    pltpu.sync_copy(table_hbm, table_v)
    pltpu.sync_copy(idx_hbm.at[pl.ds(row * E + base, CH_G)],
                    idx_v.at[pl.ds(0, CH_G)])

    @pl.when(last)
    def _copy_tail():
        pltpu.sync_copy(idx_hbm.at[pl.ds(row * E + TAIL_G_OFF, TAIL_G)],
                        idx_v.at[pl.ds(CH_G, TAIL_G)])

    def body(i, carry):
        for k in range(UNROLL):
            off = i * (LANES * UNROLL) + k * LANES
            idx = idx_v[pl.ds(off, LANES)]
            out_v[pl.ds(off, LANES)] = plsc.load_gather(table_v, [idx])
        return carry

    lax.fori_loop(0, CH_G // (LANES * UNROLL), body, 0)

    @pl.when(last)
    def _gather_tail():
        lax.fori_loop(CH_G // (LANES * UNROLL),
                      (CH_G + TAIL_G) // (LANES * UNROLL), body, 0)

    pltpu.sync_copy(out_v.at[pl.ds(0, CH_G)],
                    out_hbm.at[row, pl.ds(base, CH_G)])

    @pl.when(last)
    def _flush_tail():
        pltpu.sync_copy(out_v.at[pl.ds(CH_G, TAIL_G)],
                        out_hbm.at[row, pl.ds(TAIL_G_OFF, TAIL_G)])


@functools.lru_cache(maxsize=None)
def _sc_kernels():
    mesh = plsc.VectorSubcoreMesh(core_axis_name="c", subcore_axis_name="s",
                                  num_cores=NC, num_subcores=NS)
    degree = pl.kernel(
        _degree_body,
        out_type=(jax.ShapeDtypeStruct((NC, N), jnp.float32),
                  jax.ShapeDtypeStruct((2 * E,), jnp.int32)),
        mesh=mesh,
        scratch_types=[
            pltpu.VMEM((2, CHUNK), jnp.int32),
            pltpu.VMEM((2, TAIL), jnp.int32),
            pltpu.VMEM((CHUNK,), jnp.int32),
            pltpu.VMEM((TAIL,), jnp.int32),
            pltpu.VMEM((CHUNK,), jnp.float32),
            pltpu.VMEM((N,), jnp.float32),
            pltpu.VMEM_SHARED((N,), jnp.float32),
        ],
    )
    gather = pl.kernel(
        _gather_body,
        out_type=jax.ShapeDtypeStruct((2, E), jnp.float32),
        mesh=mesh,
        scratch_types=[
            pltpu.VMEM((N,), jnp.float32),
            pltpu.VMEM((CH_G + TAIL_G,), jnp.int32),
            pltpu.VMEM((CH_G + TAIL_G,), jnp.float32),
        ],
        compiler_params=pltpu.CompilerParams(needs_layout_passes=False),
    )
    return degree, gather


def _gelu2(x):
    return x + x * lax.erf(x * 0.7071067811865476)


def _sigmoid(x):
    return 1.0 / (1.0 + jnp.exp(-x))


def _node_mlp_body(nf_ref, part_ref, tier_ref, w1_ref, wdeg_ref, wtier_ref,
                   b1_ref, w2_ref, b2_ref, w3_ref, b3_ref, out_ref):
    part = part_ref[...]
    deg = part[0:1, :] + part[1:2, :]
    deg = deg / (jnp.max(deg) + 1e-10)
    h = lax.dot_general(w1_ref[...].astype(jnp.bfloat16),
                        nf_ref[...].astype(jnp.bfloat16),
                        (((1,), (1,)), ((), ())),
                        preferred_element_type=jnp.float32)
    h = h + wdeg_ref[...] * deg
    h = h + wtier_ref[...] * (tier_ref[...] * 0.5)
    h = _gelu2(h + b1_ref[...])
    h2 = lax.dot_general(w2_ref[...].astype(jnp.bfloat16),
                         h.astype(jnp.bfloat16),
                         (((1,), (0,)), ((), ())),
                         preferred_element_type=jnp.float32)
    h2 = _gelu2(h2 + b2_ref[...])
    s = lax.dot_general(w3_ref[...].astype(jnp.bfloat16),
                        h2.astype(jnp.bfloat16),
                        (((1,), (0,)), ((), ())),
                        preferred_element_type=jnp.float32) + b3_ref[...]
    out_ref[...] = _sigmoid(s)


def _edge_mlp_body(ef_ref, imp_ref, w1_ref, wst_ref,
                   b1_ref, w2_ref, b2_ref, w3_ref, b3_ref, out_ref):
    h = lax.dot_general(w1_ref[...].astype(jnp.bfloat16),
                        ef_ref[...],
                        (((1,), (0,)), ((), ())),
                        preferred_element_type=jnp.float32)
    h = h + lax.dot_general(wst_ref[...].astype(jnp.bfloat16),
                            imp_ref[...].astype(jnp.bfloat16),
                            (((1,), (0,)), ((), ())),
                            preferred_element_type=jnp.float32)
    h = _gelu2(h + b1_ref[...])
    h2 = lax.dot_general(w2_ref[...].astype(jnp.bfloat16),
                         h.astype(jnp.bfloat16),
                         (((1,), (0,)), ((), ())),
                         preferred_element_type=jnp.float32)
    h2 = _gelu2(h2 + b2_ref[...])
    s = lax.dot_general(w3_ref[...].astype(jnp.bfloat16),
                        h2.astype(jnp.bfloat16),
                        (((1,), (0,)), ((), ())),
                        preferred_element_type=jnp.float32) + b3_ref[...]
    out_ref[...] = _sigmoid(s)[None]


EB = 12800
EBLOCKS = E // EB


def _full(shape):
    return pl.BlockSpec(shape, lambda i: (0,) * len(shape))


def kernel(node_features, edge_index, edge_features, node_tiers,
           nW1, nb1, nW2, nb2, nW3, nb3,
           eW1, eb1, eW2, eb2, eW3, eb3):
    degree_kernel, gather_kernel = _sc_kernels()
    partials, flat_idx = degree_kernel(edge_index)

    tier = node_tiers.astype(jnp.float32).reshape(1, N)
    node_row = pl.pallas_call(
        _node_mlp_body,
        out_shape=jax.ShapeDtypeStruct((1, N), jnp.float32),
    )(node_features, partials, tier,
      nW1[:, :D_NODE], nW1[:, D_NODE:D_NODE + 1], nW1[:, D_NODE + 1:],
      nb1.reshape(H, 1), nW2 * 0.5, nb2.reshape(H, 1),
      nW3 * 0.5, nb3.reshape(1, 1))

    table = node_row.reshape(N)
    imp = gather_kernel(table, flat_idx)

    ef_t = edge_features.T.astype(jnp.bfloat16)
    edge_rows = pl.pallas_call(
        _edge_mlp_body,
        out_shape=jax.ShapeDtypeStruct((EBLOCKS, 1, EB), jnp.float32),
        grid=(EBLOCKS,),
        in_specs=[
            pl.BlockSpec((D_EDGE, EB), lambda i: (0, i)),
            pl.BlockSpec((2, EB), lambda i: (0, i)),
            _full((H, D_EDGE)), _full((H, 2)),
            _full((H, 1)), _full((H, H)), _full((H, 1)),
            _full((1, H)), _full((1, 1)),
        ],
        out_specs=pl.BlockSpec((1, 1, EB), lambda i: (i, 0, 0)),
        compiler_params=pltpu.CompilerParams(
            dimension_semantics=("arbitrary",)),
    )(ef_t, imp,
      eW1[:, :D_EDGE], eW1[:, D_EDGE:],
      eb1.reshape(H, 1), eW2 * 0.5, eb2.reshape(H, 1),
      eW3 * 0.5, eb3.reshape(1, 1))

    return table, edge_rows.reshape(E)


# speedup vs baseline: 1.0815x; 1.0815x over previous
import functools

import jax
import jax.numpy as jnp
from jax import lax
from jax.experimental import pallas as pl
from jax.experimental.pallas import tpu as pltpu
from jax.experimental.pallas import tpu_sc as plsc

N = 10000
E = 320000
D_NODE = 128
D_EDGE = 16
H = 64

NC = 2
NS = 16
LANES = 16
NW = NC * NS
CHUNK = 9984
TAIL = E - NW * CHUNK
TAIL_OFF = NW * CHUNK
UNROLL = 8
CH_G = 19968
TAIL_G = E - NS * CH_G
TAIL_G_OFF = NS * CH_G


def _degree_body(idx_hbm, out_hbm, flat_hbm, idx_v, idx_t, idx_r, idx_rt,
                 ones_v, zeros_v, shared):
    c = lax.axis_index("c")
    s = lax.axis_index("s")
    wid = s * NC + c
    base = wid * CHUNK
    last = wid == NW - 1
    pltpu.sync_copy(idx_hbm.at[:, pl.ds(base, CHUNK)], idx_v)

    @pl.when(last)
    def _copy_tail():
        pltpu.sync_copy(idx_hbm.at[:, pl.ds(TAIL_OFF, TAIL)], idx_t)

    zeros = jnp.zeros((LANES,), jnp.float32)
    ones = jnp.ones((LANES,), jnp.float32)

    @pl.when(s == 0)
    def _zero_shared():
        def zbody(i, carry):
            zeros_v[pl.ds(i * LANES, LANES)] = zeros
            return carry
        lax.fori_loop(0, N // LANES, zbody, 0)
        pltpu.sync_copy(zeros_v, shared)

    def fbody(i, carry):
        ones_v[pl.ds(i * LANES, LANES)] = ones
        return carry
    lax.fori_loop(0, CHUNK // LANES, fbody, 0)

    def make_extract(src, dst, r):
        def xbody(i, carry):
            for k in range(UNROLL):
                off = i * (LANES * UNROLL) + k * LANES
                dst[pl.ds(off, LANES)] = (
                    src[r, pl.ds(off, LANES)].reshape((LANES,)))
            return carry
        return xbody

    plsc.subcore_barrier()
    for r in range(2):
        lax.fori_loop(0, CHUNK // (LANES * UNROLL),
                      make_extract(idx_v, idx_r, r), 0)
        pltpu.sync_copy(idx_r, flat_hbm.at[pl.ds(r * E + base, CHUNK)])
        pltpu.sync_copy(ones_v, shared.at[idx_r], add=True)

    @pl.when(last)
    def _scatter_tail():
        for r in range(2):
            lax.fori_loop(0, TAIL // (LANES * UNROLL),
                          make_extract(idx_t, idx_rt, r), 0)
            pltpu.sync_copy(idx_rt,
                            flat_hbm.at[pl.ds(r * E + TAIL_OFF, TAIL)])
            pltpu.sync_copy(ones_v.at[pl.ds(0, TAIL)],
                            shared.at[idx_rt], add=True)

    plsc.subcore_barrier()

    @pl.when(s == 0)
    def _flush():
        pltpu.sync_copy(shared, out_hbm.at[c])


def _gather_body(table_hbm, idx_hbm, out_hbm, table_v, idx_v, out_v):
    c = lax.axis_index("c")
    s = lax.axis_index("s")
    wid = s * NC + c
    row = wid // NS
    t = wid % NS
    base = t * CH_G
    last = t == NS - 1
    pltpu.sync_copy(table_hbm, table_v)
    pltpu.sync_copy(idx_hbm.at[pl.ds(row * E + base, CH_G)],
                    idx_v.at[pl.ds(0, CH_G)])

    @pl.when(last)
    def _copy_tail():
        pltpu.sync_copy(idx_hbm.at[pl.ds(row * E + TAIL_G_OFF, TAIL_G)],
                        idx_v.at[pl.ds(CH_G, TAIL_G)])

    def body(i, carry):
        for k in range(UNROLL):
            off = i * (LANES * UNROLL) + k * LANES
            idx = idx_v[pl.ds(off, LANES)]
            out_v[pl.ds(off, LANES)] = plsc.load_gather(table_v, [idx])
        return carry

    lax.fori_loop(0, CH_G // (LANES * UNROLL), body, 0)

    @pl.when(last)
    def _gather_tail():
        lax.fori_loop(CH_G // (LANES * UNROLL),
                      (CH_G + TAIL_G) // (LANES * UNROLL), body, 0)

    pltpu.sync_copy(out_v.at[pl.ds(0, CH_G)],
                    out_hbm.at[row, pl.ds(base, CH_G)])

    @pl.when(last)
    def _flush_tail():
        pltpu.sync_copy(out_v.at[pl.ds(CH_G, TAIL_G)],
                        out_hbm.at[row, pl.ds(TAIL_G_OFF, TAIL_G)])


@functools.lru_cache(maxsize=None)
def _sc_kernels():
    mesh = plsc.VectorSubcoreMesh(core_axis_name="c", subcore_axis_name="s",
                                  num_cores=NC, num_subcores=NS)
    degree = pl.kernel(
        _degree_body,
        out_type=(jax.ShapeDtypeStruct((NC, N), jnp.float32),
                  jax.ShapeDtypeStruct((2 * E,), jnp.int32)),
        mesh=mesh,
        scratch_types=[
            pltpu.VMEM((2, CHUNK), jnp.int32),
            pltpu.VMEM((2, TAIL), jnp.int32),
            pltpu.VMEM((CHUNK,), jnp.int32),
            pltpu.VMEM((TAIL,), jnp.int32),
            pltpu.VMEM((CHUNK,), jnp.float32),
            pltpu.VMEM((N,), jnp.float32),
            pltpu.VMEM_SHARED((N,), jnp.float32),
        ],
    )
    gather = pl.kernel(
        _gather_body,
        out_type=jax.ShapeDtypeStruct((2, E), jnp.float32),
        mesh=mesh,
        scratch_types=[
            pltpu.VMEM((N,), jnp.float32),
            pltpu.VMEM((CH_G + TAIL_G,), jnp.int32),
            pltpu.VMEM((CH_G + TAIL_G,), jnp.float32),
        ],
        compiler_params=pltpu.CompilerParams(needs_layout_passes=False),
    )
    return degree, gather


def _gelu2(x):
    return x + x * lax.erf(x * 0.7071067811865476)


def _sigmoid(x):
    return 1.0 / (1.0 + jnp.exp(-x))


def _node_mlp_body(nf_ref, part_ref, tier_ref, w1_ref, wdeg_ref, wtier_ref,
                   b1_ref, w2_ref, b2_ref, w3_ref, b3_ref, out_ref):
    part = part_ref[...]
    deg = part[0:1, :] + part[1:2, :]
    deg = deg / (jnp.max(deg) + 1e-10)
    h = lax.dot_general(w1_ref[...].astype(jnp.bfloat16),
                        nf_ref[...].astype(jnp.bfloat16),
                        (((1,), (1,)), ((), ())),
                        preferred_element_type=jnp.float32)
    h = h + wdeg_ref[...] * deg
    h = h + wtier_ref[...] * (tier_ref[...] * 0.5)
    h = _gelu2(h + b1_ref[...])
    h2 = lax.dot_general(w2_ref[...].astype(jnp.bfloat16),
                         h.astype(jnp.bfloat16),
                         (((1,), (0,)), ((), ())),
                         preferred_element_type=jnp.float32)
    h2 = _gelu2(h2 + b2_ref[...])
    s = lax.dot_general(w3_ref[...].astype(jnp.bfloat16),
                        h2.astype(jnp.bfloat16),
                        (((1,), (0,)), ((), ())),
                        preferred_element_type=jnp.float32) + b3_ref[...]
    out_ref[...] = _sigmoid(s)


def _edge_mlp_body(ef_ref, imp_ref, w1_ref, wst_ref,
                   b1_ref, w2_ref, b2_ref, w3_ref, b3_ref, out_ref):
    x = jnp.concatenate([ef_ref[...], imp_ref[...].astype(jnp.bfloat16)],
                        axis=0)
    w1 = jnp.concatenate([w1_ref[...], wst_ref[...]], axis=1)
    h = lax.dot_general(w1.astype(jnp.bfloat16), x,
                        (((1,), (0,)), ((), ())),
                        preferred_element_type=jnp.float32)
    h = _gelu2(h + b1_ref[...])
    h2 = lax.dot_general(w2_ref[...].astype(jnp.bfloat16),
                         h.astype(jnp.bfloat16),
                         (((1,), (0,)), ((), ())),
                         preferred_element_type=jnp.float32)
    h2 = _gelu2(h2 + b2_ref[...])
    s = lax.dot_general(w3_ref[...].astype(jnp.bfloat16),
                        h2.astype(jnp.bfloat16),
                        (((1,), (0,)), ((), ())),
                        preferred_element_type=jnp.float32) + b3_ref[...]
    out_ref[...] = _sigmoid(s)[None]


EB = 12800
EBLOCKS = E // EB


def _full(shape):
    return pl.BlockSpec(shape, lambda i: (0,) * len(shape))


def kernel(node_features, edge_index, edge_features, node_tiers,
           nW1, nb1, nW2, nb2, nW3, nb3,
           eW1, eb1, eW2, eb2, eW3, eb3):
    degree_kernel, gather_kernel = _sc_kernels()
    partials, flat_idx = degree_kernel(edge_index)

    tier = node_tiers.astype(jnp.float32).reshape(1, N)
    node_row = pl.pallas_call(
        _node_mlp_body,
        out_shape=jax.ShapeDtypeStruct((1, N), jnp.float32),
    )(node_features, partials, tier,
      nW1[:, :D_NODE], nW1[:, D_NODE:D_NODE + 1], nW1[:, D_NODE + 1:],
      nb1.reshape(H, 1), nW2 * 0.5, nb2.reshape(H, 1),
      nW3 * 0.5, nb3.reshape(1, 1))

    table = node_row.reshape(N)
    imp = gather_kernel(table, flat_idx)

    ef_t = edge_features.T.astype(jnp.bfloat16)
    edge_rows = pl.pallas_call(
        _edge_mlp_body,
        out_shape=jax.ShapeDtypeStruct((EBLOCKS, 1, EB), jnp.float32),
        grid=(EBLOCKS,),
        in_specs=[
            pl.BlockSpec((D_EDGE, EB), lambda i: (0, i)),
            pl.BlockSpec((2, EB), lambda i: (0, i)),
            _full((H, D_EDGE)), _full((H, 2)),
            _full((H, 1)), _full((H, H)), _full((H, 1)),
            _full((1, H)), _full((1, 1)),
        ],
        out_specs=pl.BlockSpec((1, 1, EB), lambda i: (i, 0, 0)),
        compiler_params=pltpu.CompilerParams(
            dimension_semantics=("arbitrary",)),
    )(ef_t, imp,
      eW1[:, :D_EDGE], eW1[:, D_EDGE:],
      eb1.reshape(H, 1), eW2 * 0.5, eb2.reshape(H, 1),
      eW3 * 0.5, eb3.reshape(1, 1))

    return table, edge_rows.reshape(E)


# speedup vs baseline: 1.1902x; 1.1005x over previous
import functools

import jax
import jax.numpy as jnp
from jax import lax
from jax.experimental import pallas as pl
from jax.experimental.pallas import tpu as pltpu
from jax.experimental.pallas import tpu_sc as plsc

N = 10000
E = 320000
D_NODE = 128
D_EDGE = 16
H = 64

NC = 2
NS = 16
LANES = 16
NW = NC * NS
CHUNK = 9984
TAIL = E - NW * CHUNK
TAIL_OFF = NW * CHUNK
UNROLL = 8
CH_G = 19968
TAIL_G = E - NS * CH_G
TAIL_G_OFF = NS * CH_G


def _degree_body(idx_hbm, out_hbm, flat_hbm, idx_v, idx_t, idx_r, idx_rt,
                 ones_v, zeros_v, shared):
    c = lax.axis_index("c")
    s = lax.axis_index("s")
    wid = s * NC + c
    base = wid * CHUNK
    last = wid == NW - 1
    pltpu.sync_copy(idx_hbm.at[:, pl.ds(base, CHUNK)], idx_v)

    @pl.when(last)
    def _copy_tail():
        pltpu.sync_copy(idx_hbm.at[:, pl.ds(TAIL_OFF, TAIL)], idx_t)

    zeros = jnp.zeros((LANES,), jnp.float32)
    ones = jnp.ones((LANES,), jnp.float32)

    @pl.when(s == 0)
    def _zero_shared():
        def zbody(i, carry):
            zeros_v[pl.ds(i * LANES, LANES)] = zeros
            return carry
        lax.fori_loop(0, N // LANES, zbody, 0)
        pltpu.sync_copy(zeros_v, shared)

    def fbody(i, carry):
        ones_v[pl.ds(i * LANES, LANES)] = ones
        return carry
    lax.fori_loop(0, CHUNK // LANES, fbody, 0)

    def make_extract(src, dst, r):
        def xbody(i, carry):
            for k in range(UNROLL):
                off = i * (LANES * UNROLL) + k * LANES
                dst[pl.ds(off, LANES)] = (
                    src[r, pl.ds(off, LANES)].reshape((LANES,)))
            return carry
        return xbody

    plsc.subcore_barrier()
    for r in range(2):
        lax.fori_loop(0, CHUNK // (LANES * UNROLL),
                      make_extract(idx_v, idx_r, r), 0)
        pltpu.sync_copy(idx_r, flat_hbm.at[pl.ds(r * E + base, CHUNK)])
        pltpu.sync_copy(ones_v, shared.at[idx_r], add=True)

    @pl.when(last)
    def _scatter_tail():
        for r in range(2):
            lax.fori_loop(0, TAIL // (LANES * UNROLL),
                          make_extract(idx_t, idx_rt, r), 0)
            pltpu.sync_copy(idx_rt,
                            flat_hbm.at[pl.ds(r * E + TAIL_OFF, TAIL)])
            pltpu.sync_copy(ones_v.at[pl.ds(0, TAIL)],
                            shared.at[idx_rt], add=True)

    plsc.subcore_barrier()

    @pl.when(s == 0)
    def _flush():
        pltpu.sync_copy(shared, out_hbm.at[c])


def _gather_body(table_hbm, idx_hbm, out_hbm, table_v, idx_v, out_v):
    c = lax.axis_index("c")
    s = lax.axis_index("s")
    wid = s * NC + c
    row = wid // NS
    t = wid % NS
    base = t * CH_G
    last = t == NS - 1
    pltpu.sync_copy(table_hbm, table_v)
    pltpu.sync_copy(idx_hbm.at[pl.ds(row * E + base, CH_G)],
                    idx_v.at[pl.ds(0, CH_G)])

    @pl.when(last)
    def _copy_tail():
        pltpu.sync_copy(idx_hbm.at[pl.ds(row * E + TAIL_G_OFF, TAIL_G)],
                        idx_v.at[pl.ds(CH_G, TAIL_G)])

    def body(i, carry):
        for k in range(UNROLL):
            off = i * (LANES * UNROLL) + k * LANES
            idx = idx_v[pl.ds(off, LANES)]
            out_v[pl.ds(off, LANES)] = plsc.load_gather(table_v, [idx])
        return carry

    lax.fori_loop(0, CH_G // (LANES * UNROLL), body, 0)

    @pl.when(last)
    def _gather_tail():
        lax.fori_loop(CH_G // (LANES * UNROLL),
                      (CH_G + TAIL_G) // (LANES * UNROLL), body, 0)

    pltpu.sync_copy(out_v.at[pl.ds(0, CH_G)],
                    out_hbm.at[row, pl.ds(base, CH_G)])

    @pl.when(last)
    def _flush_tail():
        pltpu.sync_copy(out_v.at[pl.ds(CH_G, TAIL_G)],
                        out_hbm.at[row, pl.ds(TAIL_G_OFF, TAIL_G)])


@functools.lru_cache(maxsize=None)
def _sc_kernels():
    mesh = plsc.VectorSubcoreMesh(core_axis_name="c", subcore_axis_name="s",
                                  num_cores=NC, num_subcores=NS)
    degree = pl.kernel(
        _degree_body,
        out_type=(jax.ShapeDtypeStruct((NC, N), jnp.float32),
                  jax.ShapeDtypeStruct((2 * E,), jnp.int32)),
        mesh=mesh,
        scratch_types=[
            pltpu.VMEM((2, CHUNK), jnp.int32),
            pltpu.VMEM((2, TAIL), jnp.int32),
            pltpu.VMEM((CHUNK,), jnp.int32),
            pltpu.VMEM((TAIL,), jnp.int32),
            pltpu.VMEM((CHUNK,), jnp.float32),
            pltpu.VMEM((N,), jnp.float32),
            pltpu.VMEM_SHARED((N,), jnp.float32),
        ],
    )
    gather = pl.kernel(
        _gather_body,
        out_type=jax.ShapeDtypeStruct((2, E), jnp.float32),
        mesh=mesh,
        scratch_types=[
            pltpu.VMEM((N,), jnp.float32),
            pltpu.VMEM((CH_G + TAIL_G,), jnp.int32),
            pltpu.VMEM((CH_G + TAIL_G,), jnp.float32),
        ],
        compiler_params=pltpu.CompilerParams(needs_layout_passes=False),
    )
    return degree, gather


def _gelu2(x):
    return x + x * lax.erf(x * 0.7071067811865476)


def _sigmoid(x):
    return 1.0 / (1.0 + jnp.exp(-x))


def _node_mlp_body(nf_ref, part_ref, tier_ref, w1_ref, wdeg_ref, wtier_ref,
                   b1_ref, w2_ref, b2_ref, w3_ref, b3_ref, out_ref):
    part = part_ref[...]
    deg = part[0:1, :] + part[1:2, :]
    deg = deg / (jnp.max(deg) + 1e-10)
    h = lax.dot_general(w1_ref[...].astype(jnp.bfloat16),
                        nf_ref[...].astype(jnp.bfloat16),
                        (((1,), (1,)), ((), ())),
                        preferred_element_type=jnp.float32)
    h = h + wdeg_ref[...] * deg
    h = h + wtier_ref[...] * (tier_ref[...] * 0.5)
    h = _gelu2(h + b1_ref[...])
    h2 = lax.dot_general(w2_ref[...].astype(jnp.bfloat16),
                         h.astype(jnp.bfloat16),
                         (((1,), (0,)), ((), ())),
                         preferred_element_type=jnp.float32)
    h2 = _gelu2(h2 + b2_ref[...])
    s = lax.dot_general(w3_ref[...].astype(jnp.bfloat16),
                        h2.astype(jnp.bfloat16),
                        (((1,), (0,)), ((), ())),
                        preferred_element_type=jnp.float32) + b3_ref[...]
    out_ref[...] = _sigmoid(s)


def _edge_mlp_body(ef_ref, imp_ref, w1_ref, wst_ref,
                   b1_ref, w2_ref, b2_ref, w3_ref, b3_ref, out_ref):
    x = jnp.concatenate([ef_ref[...], imp_ref[...].astype(jnp.bfloat16)],
                        axis=0)
    w1 = jnp.concatenate([w1_ref[...], wst_ref[...]], axis=1)
    h = lax.dot_general(w1.astype(jnp.bfloat16), x,
                        (((1,), (0,)), ((), ())),
                        preferred_element_type=jnp.float32)
    h = _gelu2((h + b1_ref[...]).astype(jnp.bfloat16))
    h2 = lax.dot_general(w2_ref[...].astype(jnp.bfloat16), h,
                         (((1,), (0,)), ((), ())),
                         preferred_element_type=jnp.float32)
    h2 = _gelu2((h2 + b2_ref[...]).astype(jnp.bfloat16))
    s = lax.dot_general(w3_ref[...].astype(jnp.bfloat16), h2,
                        (((1,), (0,)), ((), ())),
                        preferred_element_type=jnp.float32) + b3_ref[...]
    out_ref[...] = _sigmoid(s)[None]


EB = 12800
EBLOCKS = E // EB


def _full(shape):
    return pl.BlockSpec(shape, lambda i: (0,) * len(shape))


def kernel(node_features, edge_index, edge_features, node_tiers,
           nW1, nb1, nW2, nb2, nW3, nb3,
           eW1, eb1, eW2, eb2, eW3, eb3):
    degree_kernel, gather_kernel = _sc_kernels()
    partials, flat_idx = degree_kernel(edge_index)

    tier = node_tiers.astype(jnp.float32).reshape(1, N)
    node_row = pl.pallas_call(
        _node_mlp_body,
        out_shape=jax.ShapeDtypeStruct((1, N), jnp.float32),
    )(node_features, partials, tier,
      nW1[:, :D_NODE], nW1[:, D_NODE:D_NODE + 1], nW1[:, D_NODE + 1:],
      nb1.reshape(H, 1), nW2 * 0.5, nb2.reshape(H, 1),
      nW3 * 0.5, nb3.reshape(1, 1))

    table = node_row.reshape(N)
    imp = gather_kernel(table, flat_idx)

    ef_t = edge_features.T.astype(jnp.bfloat16)
    edge_rows = pl.pallas_call(
        _edge_mlp_body,
        out_shape=jax.ShapeDtypeStruct((EBLOCKS, 1, EB), jnp.float32),
        grid=(EBLOCKS,),
        in_specs=[
            pl.BlockSpec((D_EDGE, EB), lambda i: (0, i)),
            pl.BlockSpec((2, EB), lambda i: (0, i)),
            _full((H, D_EDGE)), _full((H, 2)),
            _full((H, 1)), _full((H, H)), _full((H, 1)),
            _full((1, H)), _full((1, 1)),
        ],
        out_specs=pl.BlockSpec((1, 1, EB), lambda i: (i, 0, 0)),
        compiler_params=pltpu.CompilerParams(
            dimension_semantics=("arbitrary",)),
    )(ef_t, imp,
      eW1[:, :D_EDGE], eW1[:, D_EDGE:],
      eb1.reshape(H, 1), eW2 * 0.5, eb2.reshape(H, 1),
      eW3 * 0.5, eb3.reshape(1, 1))

    return table, edge_rows.reshape(E)


# speedup vs baseline: 1.1929x; 1.0023x over previous
import functools

import jax
import jax.numpy as jnp
from jax import lax
from jax.experimental import pallas as pl
from jax.experimental.pallas import tpu as pltpu
from jax.experimental.pallas import tpu_sc as plsc

N = 10000
E = 320000
D_NODE = 128
D_EDGE = 16
H = 64

NC = 2
NS = 16
LANES = 16
NW = NC * NS
CHUNK = 9984
TAIL = E - NW * CHUNK
TAIL_OFF = NW * CHUNK
UNROLL = 8
CH_G = 19968
TAIL_G = E - NS * CH_G
TAIL_G_OFF = NS * CH_G


def _degree_body(idx_hbm, out_hbm, flat_hbm, idx_v, idx_t, idx_r, idx_rt,
                 ones_v, zeros_v, shared):
    c = lax.axis_index("c")
    s = lax.axis_index("s")
    wid = s * NC + c
    base = wid * CHUNK
    last = wid == NW - 1
    pltpu.sync_copy(idx_hbm.at[:, pl.ds(base, CHUNK)], idx_v)

    @pl.when(last)
    def _copy_tail():
        pltpu.sync_copy(idx_hbm.at[:, pl.ds(TAIL_OFF, TAIL)], idx_t)

    zeros = jnp.zeros((LANES,), jnp.float32)
    ones = jnp.ones((LANES,), jnp.float32)

    @pl.when(s == 0)
    def _zero_shared():
        def zbody(i, carry):
            zeros_v[pl.ds(i * LANES, LANES)] = zeros
            return carry
        lax.fori_loop(0, N // LANES, zbody, 0)
        pltpu.sync_copy(zeros_v, shared)

    def fbody(i, carry):
        ones_v[pl.ds(i * LANES, LANES)] = ones
        return carry
    lax.fori_loop(0, CHUNK // LANES, fbody, 0)

    def make_extract(src, dst, r):
        def xbody(i, carry):
            for k in range(UNROLL):
                off = i * (LANES * UNROLL) + k * LANES
                dst[pl.ds(off, LANES)] = (
                    src[r, pl.ds(off, LANES)].reshape((LANES,)))
            return carry
        return xbody

    plsc.subcore_barrier()
    for r in range(2):
        lax.fori_loop(0, CHUNK // (LANES * UNROLL),
                      make_extract(idx_v, idx_r, r), 0)
        pltpu.sync_copy(idx_r, flat_hbm.at[pl.ds(r * E + base, CHUNK)])
        pltpu.sync_copy(ones_v, shared.at[idx_r], add=True)

    @pl.when(last)
    def _scatter_tail():
        for r in range(2):
            lax.fori_loop(0, TAIL // (LANES * UNROLL),
                          make_extract(idx_t, idx_rt, r), 0)
            pltpu.sync_copy(idx_rt,
                            flat_hbm.at[pl.ds(r * E + TAIL_OFF, TAIL)])
            pltpu.sync_copy(ones_v.at[pl.ds(0, TAIL)],
                            shared.at[idx_rt], add=True)

    plsc.subcore_barrier()

    @pl.when(s == 0)
    def _flush():
        pltpu.sync_copy(shared, out_hbm.at[c])


def _gather_body(table_hbm, idx_hbm, out_hbm, table_v, idx_v, out_v):
    c = lax.axis_index("c")
    s = lax.axis_index("s")
    wid = s * NC + c
    row = wid // NS
    t = wid % NS
    base = t * CH_G
    last = t == NS - 1
    pltpu.sync_copy(table_hbm, table_v)
    pltpu.sync_copy(idx_hbm.at[pl.ds(row * E + base, CH_G)],
                    idx_v.at[pl.ds(0, CH_G)])

    @pl.when(last)
    def _copy_tail():
        pltpu.sync_copy(idx_hbm.at[pl.ds(row * E + TAIL_G_OFF, TAIL_G)],
                        idx_v.at[pl.ds(CH_G, TAIL_G)])

    def body(i, carry):
        for k in range(UNROLL):
            off = i * (LANES * UNROLL) + k * LANES
            idx = idx_v[pl.ds(off, LANES)]
            out_v[pl.ds(off, LANES)] = plsc.load_gather(table_v, [idx])
        return carry

    lax.fori_loop(0, CH_G // (LANES * UNROLL), body, 0)

    @pl.when(last)
    def _gather_tail():
        lax.fori_loop(CH_G // (LANES * UNROLL),
                      (CH_G + TAIL_G) // (LANES * UNROLL), body, 0)

    pltpu.sync_copy(out_v.at[pl.ds(0, CH_G)],
                    out_hbm.at[row, pl.ds(base, CH_G)])

    @pl.when(last)
    def _flush_tail():
        pltpu.sync_copy(out_v.at[pl.ds(CH_G, TAIL_G)],
                        out_hbm.at[row, pl.ds(TAIL_G_OFF, TAIL_G)])


@functools.lru_cache(maxsize=None)
def _sc_kernels():
    mesh = plsc.VectorSubcoreMesh(core_axis_name="c", subcore_axis_name="s",
                                  num_cores=NC, num_subcores=NS)
    degree = pl.kernel(
        _degree_body,
        out_type=(jax.ShapeDtypeStruct((NC, N), jnp.float32),
                  jax.ShapeDtypeStruct((2 * E,), jnp.int32)),
        mesh=mesh,
        scratch_types=[
            pltpu.VMEM((2, CHUNK), jnp.int32),
            pltpu.VMEM((2, TAIL), jnp.int32),
            pltpu.VMEM((CHUNK,), jnp.int32),
            pltpu.VMEM((TAIL,), jnp.int32),
            pltpu.VMEM((CHUNK,), jnp.float32),
            pltpu.VMEM((N,), jnp.float32),
            pltpu.VMEM_SHARED((N,), jnp.float32),
        ],
    )
    gather = pl.kernel(
        _gather_body,
        out_type=jax.ShapeDtypeStruct((2, E), jnp.float32),
        mesh=mesh,
        scratch_types=[
            pltpu.VMEM((N,), jnp.float32),
            pltpu.VMEM((CH_G + TAIL_G,), jnp.int32),
            pltpu.VMEM((CH_G + TAIL_G,), jnp.float32),
        ],
        compiler_params=pltpu.CompilerParams(needs_layout_passes=False),
    )
    return degree, gather


def _gelu2(x):
    return x + x * lax.erf(x * 0.7071067811865476)


def _sigmoid(x):
    return 1.0 / (1.0 + jnp.exp(-x))


def _node_mlp_body(nf_ref, part_ref, tier_ref, w1_ref, wdeg_ref, wtier_ref,
                   b1_ref, w2_ref, b2_ref, w3_ref, b3_ref, out_ref):
    part = part_ref[...]
    deg = part[0:1, :] + part[1:2, :]
    deg = deg / (jnp.max(deg) + 1e-10)
    h = lax.dot_general(w1_ref[...].astype(jnp.bfloat16),
                        nf_ref[...].astype(jnp.bfloat16),
                        (((1,), (1,)), ((), ())),
                        preferred_element_type=jnp.float32)
    h = h + wdeg_ref[...] * deg
    h = h + wtier_ref[...] * (tier_ref[...] * 0.5)
    h = _gelu2((h + b1_ref[...]).astype(jnp.bfloat16))
    h2 = lax.dot_general(w2_ref[...].astype(jnp.bfloat16), h,
                         (((1,), (0,)), ((), ())),
                         preferred_element_type=jnp.float32)
    h2 = _gelu2((h2 + b2_ref[...]).astype(jnp.bfloat16))
    s = lax.dot_general(w3_ref[...].astype(jnp.bfloat16), h2,
                        (((1,), (0,)), ((), ())),
                        preferred_element_type=jnp.float32) + b3_ref[...]
    out_ref[...] = _sigmoid(s)


def _edge_mlp_body(ef_ref, imp_ref, w1_ref, wst_ref,
                   b1_ref, w2_ref, b2_ref, w3_ref, b3_ref, out_ref):
    x = jnp.concatenate([ef_ref[...], imp_ref[...].astype(jnp.bfloat16)],
                        axis=0)
    w1 = jnp.concatenate([w1_ref[...], wst_ref[...]], axis=1)
    h = lax.dot_general(w1.astype(jnp.bfloat16), x,
                        (((1,), (0,)), ((), ())),
                        preferred_element_type=jnp.float32)
    h = _gelu2((h + b1_ref[...]).astype(jnp.bfloat16))
    h2 = lax.dot_general(w2_ref[...].astype(jnp.bfloat16), h,
                         (((1,), (0,)), ((), ())),
                         preferred_element_type=jnp.float32)
    h2 = _gelu2((h2 + b2_ref[...]).astype(jnp.bfloat16))
    s = lax.dot_general(w3_ref[...].astype(jnp.bfloat16), h2,
                        (((1,), (0,)), ((), ())),
                        preferred_element_type=jnp.float32) + b3_ref[...]
    out_ref[...] = _sigmoid(s)[None]


EB = 12800
EBLOCKS = E // EB


def _full(shape):
    return pl.BlockSpec(shape, lambda i: (0,) * len(shape))


def kernel(node_features, edge_index, edge_features, node_tiers,
           nW1, nb1, nW2, nb2, nW3, nb3,
           eW1, eb1, eW2, eb2, eW3, eb3):
    degree_kernel, gather_kernel = _sc_kernels()
    partials, flat_idx = degree_kernel(edge_index)

    tier = node_tiers.astype(jnp.float32).reshape(1, N)
    node_row = pl.pallas_call(
        _node_mlp_body,
        out_shape=jax.ShapeDtypeStruct((1, N), jnp.float32),
    )(node_features, partials, tier,
      nW1[:, :D_NODE], nW1[:, D_NODE:D_NODE + 1], nW1[:, D_NODE + 1:],
      nb1.reshape(H, 1), nW2 * 0.5, nb2.reshape(H, 1),
      nW3 * 0.5, nb3.reshape(1, 1))

    table = node_row.reshape(N)
    imp = gather_kernel(table, flat_idx)

    ef_t = edge_features.T.astype(jnp.bfloat16)
    edge_rows = pl.pallas_call(
        _edge_mlp_body,
        out_shape=jax.ShapeDtypeStruct((EBLOCKS, 1, EB), jnp.float32),
        grid=(EBLOCKS,),
        in_specs=[
            pl.BlockSpec((D_EDGE, EB), lambda i: (0, i)),
            pl.BlockSpec((2, EB), lambda i: (0, i)),
            _full((H, D_EDGE)), _full((H, 2)),
            _full((H, 1)), _full((H, H)), _full((H, 1)),
            _full((1, H)), _full((1, 1)),
        ],
        out_specs=pl.BlockSpec((1, 1, EB), lambda i: (i, 0, 0)),
        compiler_params=pltpu.CompilerParams(
            dimension_semantics=("arbitrary",)),
    )(ef_t, imp,
      eW1[:, :D_EDGE], eW1[:, D_EDGE:],
      eb1.reshape(H, 1), eW2 * 0.5, eb2.reshape(H, 1),
      eW3 * 0.5, eb3.reshape(1, 1))

    return table, edge_rows.reshape(E)
